# baseline (speedup 1.0000x reference)
import functools

import jax
import jax.numpy as jnp
from jax import lax
from jax.experimental import pallas as pl
from jax.experimental.pallas import tpu as pltpu
from jax.experimental.pallas import tpu_sc as plsc

NC = 2
NS = 16
L = 16
NW = NC * NS

HROWS = 512
CROWS = 16
UNROLL = 4


def _sc_hist(x2):
    nrows, _ = x2.shape
    pw = nrows // NW
    nchunk = pw // CROWS
    mesh = plsc.VectorSubcoreMesh(core_axis_name="c", subcore_axis_name="s")

    @functools.partial(
        pl.kernel,
        mesh=mesh,
        out_type=jax.ShapeDtypeStruct((NW, HROWS, 128), jnp.int32),
        scratch_types=[
            pltpu.VMEM((CROWS, 1024), jnp.float32),
            pltpu.VMEM((CROWS, 1024), jnp.float32),
            pltpu.VMEM((HROWS, 128), jnp.int32),
            pltpu.SemaphoreType.DMA,
            pltpu.SemaphoreType.DMA,
        ],
        compiler_params=pltpu.CompilerParams(needs_layout_passes=False),
    )
    def k(x_hbm, out_hbm, buf0, buf1, hist, sem0, sem1):
        wid = lax.axis_index("s") * NC + lax.axis_index("c")
        base = wid * pw
        bufs, sems = (buf0, buf1), (sem0, sem1)

        def cp(j, b, s):
            return pltpu.make_async_copy(
                x_hbm.at[pl.ds(base + j * CROWS, CROWS), :], b, s)

        cp(0, bufs[0], sems[0]).start()

        zeros = jnp.zeros((L,), jnp.int32)
        ones = jnp.ones((L,), jnp.int32)

        def zbody(i, c):
            for t in range(UNROLL):
                ii = i * UNROLL + t
                hist[ii >> 3, pl.ds((ii & 7) * L, L)] = zeros
            return c

        lax.fori_loop(0, HROWS * 8 // UNROLL, zbody, 0)

        for j in range(nchunk):
            if j + 1 < nchunk:
                cp(j + 1, bufs[(j + 1) % 2], sems[(j + 1) % 2]).start()
            cp(j, bufs[j % 2], sems[j % 2]).wait()
            buf = bufs[j % 2]

            @plsc.parallel_loop(0, CROWS * (1024 // L), 1, unroll=UNROLL)
            def _(i):
                v = buf[i >> 6, pl.ds((i & 63) * L, L)]
                u = plsc.bitcast(v, jnp.int32)
                b = (u & jnp.int32(0x7FFFFFFF)) >> 15
                plsc.addupdate_scatter(hist, [b >> 7, b & jnp.int32(127)],
                                       ones)

        pltpu.sync_copy(hist, out_hbm.at[wid])

    return k(x2)


def _cumsum2d(h2):
    rows, lanes = h2.shape
    c = h2
    s = 1
    while s < lanes:
        c = c + jnp.concatenate(
            [jnp.zeros((rows, s), jnp.float32), c[:, : lanes - s]], axis=1)
        s *= 2
    t = c[:, lanes - 1:lanes]
    s = 1
    while s < rows:
        t = t + jnp.concatenate(
            [jnp.zeros((s, 1), jnp.float32), t[: rows - s, :]], axis=0)
        s *= 2
    pre = jnp.concatenate(
        [jnp.zeros((1, 1), jnp.float32), t[: rows - 1, :]], axis=0)
    return c + pre


def _tc_select_clamp(hist3, x2, kth):
    rows = x2.shape[0]
    blk = 1024
    nsteps = rows // blk

    def body(h_ref, x_ref, o_ref, cvs):
        i = pl.program_id(0)

        @pl.when(i == 0)
        def _():
            h = jnp.sum(h_ref[...].astype(jnp.float32), axis=0)
            c = _cumsum2d(h)
            bbin = jnp.sum((c <= jnp.float32(kth)).astype(jnp.int32))
            bits = jnp.full((8, 128), (bbin << 15) | 0x4000, jnp.int32)
            cvs[...] = lax.bitcast_convert_type(bits, jnp.float32)

        c = cvs[0, 0]
        o_ref[...] = jnp.clip(x_ref[...], -c, c)

    return pl.pallas_call(
        body,
        grid=(nsteps,),
        in_specs=[
            pl.BlockSpec((NW, HROWS, 128), lambda i: (0, 0, 0)),
            pl.BlockSpec((blk, 1024), lambda i: (i, 0)),
        ],
        out_specs=pl.BlockSpec((blk, 1024), lambda i: (i, 0)),
        out_shape=jax.ShapeDtypeStruct(x2.shape, jnp.float32),
        scratch_shapes=[pltpu.VMEM((8, 128), jnp.float32)],
    )(hist3, x2)


def kernel(x):
    n = x.size
    kth = int(round(0.99 * n)) - 1
    x2 = x.reshape(-1, 1024)

    hist = _sc_hist(x2)
    out2 = _tc_select_clamp(hist, x2, kth)
    return out2.reshape(x.shape)


# speedup vs baseline: 1.0388x; 1.0388x over previous
import functools

import jax
import jax.numpy as jnp
from jax import lax
from jax.experimental import pallas as pl
from jax.experimental.pallas import tpu as pltpu
from jax.experimental.pallas import tpu_sc as plsc

NC = 2
NS = 16
L = 16
NW = NC * NS

HROWS = 512
CROWS = 16
UNROLL = 8


def _sc_hist(x2):
    nrows, _ = x2.shape
    pw = nrows // NW
    nchunk = pw // CROWS
    mesh = plsc.VectorSubcoreMesh(core_axis_name="c", subcore_axis_name="s")

    @functools.partial(
        pl.kernel,
        mesh=mesh,
        out_type=jax.ShapeDtypeStruct((NW, HROWS, 128), jnp.int32),
        scratch_types=[
            pltpu.VMEM((CROWS, 1024), jnp.float32),
            pltpu.VMEM((CROWS, 1024), jnp.float32),
            pltpu.VMEM((HROWS, 128), jnp.int32),
            pltpu.SemaphoreType.DMA,
            pltpu.SemaphoreType.DMA,
        ],
        compiler_params=pltpu.CompilerParams(needs_layout_passes=False),
    )
    def k(x_hbm, out_hbm, buf0, buf1, hist, sem0, sem1):
        wid = lax.axis_index("s") * NC + lax.axis_index("c")
        base = wid * pw
        bufs, sems = (buf0, buf1), (sem0, sem1)

        def cp(j, b, s):
            return pltpu.make_async_copy(
                x_hbm.at[pl.ds(base + j * CROWS, CROWS), :], b, s)

        cp(0, bufs[0], sems[0]).start()

        zeros = jnp.zeros((L,), jnp.int32)
        ones = jnp.ones((L,), jnp.int32)

        def zbody(i, c):
            for t in range(UNROLL):
                ii = i * UNROLL + t
                hist[ii >> 3, pl.ds((ii & 7) * L, L)] = zeros
            return c

        lax.fori_loop(0, HROWS * 8 // UNROLL, zbody, 0)

        for j in range(nchunk):
            if j + 1 < nchunk:
                cp(j + 1, bufs[(j + 1) % 2], sems[(j + 1) % 2]).start()
            cp(j, bufs[j % 2], sems[j % 2]).wait()
            buf = bufs[j % 2]

            @plsc.parallel_loop(0, CROWS * (1024 // L), 1, unroll=UNROLL)
            def _(i):
                v = buf[i >> 6, pl.ds((i & 63) * L, L)]
                u = plsc.bitcast(v, jnp.int32)
                b = (u & jnp.int32(0x7FFFFFFF)) >> 15
                plsc.addupdate_scatter(hist, [b >> 7, b & jnp.int32(127)],
                                       ones)

        pltpu.sync_copy(hist, out_hbm.at[wid])

    return k(x2)


def _cumsum2d(h2):
    rows, lanes = h2.shape
    c = h2
    s = 1
    while s < lanes:
        c = c + jnp.concatenate(
            [jnp.zeros((rows, s), jnp.float32), c[:, : lanes - s]], axis=1)
        s *= 2
    t = c[:, lanes - 1:lanes]
    s = 1
    while s < rows:
        t = t + jnp.concatenate(
            [jnp.zeros((s, 1), jnp.float32), t[: rows - s, :]], axis=0)
        s *= 2
    pre = jnp.concatenate(
        [jnp.zeros((1, 1), jnp.float32), t[: rows - 1, :]], axis=0)
    return c + pre


def _tc_select_clamp(hist3, x2, kth):
    rows = x2.shape[0]
    blk = 2048
    nsteps = rows // blk

    def body(h_ref, x_ref, o_ref, cvs):
        i = pl.program_id(0)

        @pl.when(i == 0)
        def _():
            h = jnp.sum(h_ref[...].astype(jnp.float32), axis=0)
            c = _cumsum2d(h)
            bbin = jnp.sum((c <= jnp.float32(kth)).astype(jnp.int32))
            bits = jnp.full((8, 128), (bbin << 15) | 0x4000, jnp.int32)
            cvs[...] = lax.bitcast_convert_type(bits, jnp.float32)

        c = cvs[0, 0]
        o_ref[...] = jnp.clip(x_ref[...], -c, c)

    return pl.pallas_call(
        body,
        grid=(nsteps,),
        in_specs=[
            pl.BlockSpec((NW, HROWS, 128), lambda i: (0, 0, 0)),
            pl.BlockSpec((blk, 1024), lambda i: (i, 0)),
        ],
        out_specs=pl.BlockSpec((blk, 1024), lambda i: (i, 0)),
        out_shape=jax.ShapeDtypeStruct(x2.shape, jnp.float32),
        scratch_shapes=[pltpu.VMEM((8, 128), jnp.float32)],
    )(hist3, x2)


def kernel(x):
    n = x.size
    kth = int(round(0.99 * n)) - 1
    x2 = x.reshape(-1, 1024)

    hist = _sc_hist(x2)
    out2 = _tc_select_clamp(hist, x2, kth)
    return out2.reshape(x.shape)
